# baseline (speedup 1.0000x reference)
import functools
import math

import jax
import jax.numpy as jnp
from jax import lax
from jax.experimental import pallas as pl
from jax.experimental.pallas import tpu as pltpu
from jax.experimental.pallas import tpu_sc as plsc

_SIZE = 32000
_PAD = 0
_SMOOTH = 0.1
_CONF = 1.0 - _SMOOTH
_EPS = _SMOOTH / (_SIZE - 2)
_N = 4096
_C1 = _EPS * math.log(_EPS) * (_SIZE - 2) + _CONF * math.log(_CONF)

_NW = 32
_R_SC = 1024
_R_TC = _N - _R_SC
_RPW = _R_SC // _NW
_PICKS_PW = _N // _NW
_BM = 128
_BN = _SIZE


def _tc_kernel(t_ref, x_ref, o_ref):
    i = pl.program_id(0)

    @pl.when(i == 0)
    def _():
        o_ref[...] = jnp.zeros_like(o_ref)

    t = t_ref[...]
    rowsum = jnp.sum(x_ref[...], axis=1, keepdims=True)
    acc = jnp.sum(jnp.where(t != _PAD, -_EPS, 0.0) * rowsum)
    o_ref[...] += acc.reshape(1, 1)


def _tc_call(t32, x):
    return pl.pallas_call(
        _tc_kernel,
        grid=(_R_TC // _BM,),
        in_specs=[
            pl.BlockSpec((_BM, 1), lambda i: (i, 0)),
            pl.BlockSpec((_BM, _BN), lambda i: (i, 0)),
        ],
        out_specs=pl.BlockSpec((1, 1), lambda i: (0, 0)),
        out_shape=jax.ShapeDtypeStruct((1, 1), jnp.float32),
    )(t32.reshape(_N, 1), x)


def _sc_body(x_hbm, xflat_hbm, t_hbm, out_hbm,
             rowbuf, idxt, idx0, picks, col0s, tpick, trows, outv, sem):
    wid = lax.axis_index("c") * 16 + lax.axis_index("s")
    acc = jnp.zeros((16,), jnp.float32)

    pick_base = wid * _PICKS_PW
    pltpu.sync_copy(t_hbm.at[pl.ds(pick_base, _PICKS_PW)], tpick)
    for k in range(_PICKS_PW // 16):
        rowv = (pick_base + k * 16 + lax.iota(jnp.int32, 16)) * _SIZE
        idx0[pl.ds(k * 16, 16)] = rowv
        idxt[pl.ds(k * 16, 16)] = rowv + tpick[pl.ds(k * 16, 16)]
    pltpu.async_copy(xflat_hbm.at[idxt], picks, sem).wait()
    pltpu.async_copy(xflat_hbm.at[idx0], col0s, sem).wait()
    for k in range(_PICKS_PW // 16):
        tk = tpick[pl.ds(k * 16, 16)]
        corr = ((_EPS - _CONF) * picks[pl.ds(k * 16, 16)]
                + _EPS * col0s[pl.ds(k * 16, 16)] + _C1)
        acc = acc + jnp.where(tk != _PAD, corr, 0.0)

    row_base = _R_TC + wid * _RPW
    pltpu.sync_copy(t_hbm.at[pl.ds(row_base, _RPW)], trows)

    def _row_sum():
        def chunk_body(k, accs):
            a0, a1, a2, a3 = accs
            b = k * 128
            a0 = a0 + rowbuf[pl.ds(b, 16)] + rowbuf[pl.ds(b + 64, 16)]
            a1 = a1 + rowbuf[pl.ds(b + 16, 16)] + rowbuf[pl.ds(b + 80, 16)]
            a2 = a2 + rowbuf[pl.ds(b + 32, 16)] + rowbuf[pl.ds(b + 96, 16)]
            a3 = a3 + rowbuf[pl.ds(b + 48, 16)] + rowbuf[pl.ds(b + 112, 16)]
            return (a0, a1, a2, a3)

        z = jnp.zeros((16,), jnp.float32)
        a0, a1, a2, a3 = lax.fori_loop(0, _SIZE // 128, chunk_body,
                                       (z, z, z, z))
        return (a0 + a1) + (a2 + a3)

    def group_body(g, acc):
        tv = trows[pl.ds(g * 16, 16)]
        wv = jnp.where(tv != _PAD, jnp.float32(-_EPS), jnp.float32(0.0))
        r0 = row_base + g * 16
        for rr in range(16):
            pltpu.sync_copy(x_hbm.at[r0 + rr], rowbuf)
            acc = acc + wv[rr] * _row_sum()
        return acc

    acc = lax.fori_loop(0, _RPW // 16, group_body, acc)

    outv[...] = acc
    pltpu.sync_copy(outv, out_hbm.at[wid])


def _sc_call(x, xflat, t32):
    mesh = plsc.VectorSubcoreMesh(core_axis_name="c", subcore_axis_name="s")
    f = pl.kernel(
        _sc_body,
        mesh=mesh,
        out_type=jax.ShapeDtypeStruct((_NW, 16), jnp.float32),
        scratch_types=[
            pltpu.VMEM((_SIZE,), jnp.float32),
            pltpu.VMEM((_PICKS_PW,), jnp.int32),
            pltpu.VMEM((_PICKS_PW,), jnp.int32),
            pltpu.VMEM((_PICKS_PW,), jnp.float32),
            pltpu.VMEM((_PICKS_PW,), jnp.float32),
            pltpu.VMEM((_PICKS_PW,), jnp.int32),
            pltpu.VMEM((_RPW,), jnp.int32),
            pltpu.VMEM((16,), jnp.float32),
            pltpu.SemaphoreType.DMA,
        ],
    )
    return f(x, xflat, t32)


@jax.jit
def kernel(x, target):
    t32 = target.astype(jnp.int32)
    sc_part = _sc_call(x, x.reshape(-1), t32)
    tc_part = _tc_call(t32, x)
    return tc_part[0, 0] + jnp.sum(sc_part)
